# baseline (speedup 1.0000x reference)
import functools

import jax
import jax.numpy as jnp
from jax import lax
from jax.experimental import pallas as pl
from jax.experimental.pallas import tpu as pltpu
from jax.experimental.pallas import tpu_sc as plsc

_D = 32
_NC = 2
_NS = 16
_NW = _NC * _NS
_L = 200
_B = 4096
_LH = _L // 8
_BH = _B // 128
_NBLK = _LH * _BH * 2
_BPW = _NBLK // _NW


def _make_kernel():
    mesh = plsc.VectorSubcoreMesh(core_axis_name="c", subcore_axis_name="s")

    @functools.partial(
        pl.kernel,
        mesh=mesh,
        compiler_params=pltpu.CompilerParams(
            use_tc_tiling_on_sc=False, needs_layout_passes=False),
        out_type=jax.ShapeDtypeStruct((_L, 4, _BH, 8, 128), jnp.float32),
        scratch_types=[
            pltpu.VMEM((8, 64), jnp.int32),
            pltpu.VMEM((8, 64), jnp.int32),
            pltpu.VMEM((8, 64, _D), jnp.float32),
            pltpu.VMEM((8, 64, _D), jnp.float32),
            pltpu.VMEM((256, 64), jnp.float32),
            pltpu.VMEM((256, 64), jnp.float32),
            pltpu.VMEM((16, _D), jnp.float32),
            pltpu.SemaphoreType.DMA,
            pltpu.SemaphoreType.DMA,
            pltpu.SemaphoreType.DMA,
            pltpu.SemaphoreType.DMA,
            pltpu.SemaphoreType.DMA,
            pltpu.SemaphoreType.DMA,
        ],
    )
    def k(idx4, table, ptile, out5,
          idxb0, idxb1, rows0, rows1, trans0, trans1, pos_v,
          si0, si1, sg0, sg1, so0, so1):
        c = lax.axis_index("c")
        s = lax.axis_index("s")
        w = s * _NC + c
        hb0 = w * _BPW
        lh_first = hb0 // 64
        lh_last = (hb0 + _BPW - 1) // 64

        pltpu.sync_copy(ptile.at[pl.ds(lh_first * 8, 8)], pos_v.at[pl.ds(0, 8)])
        pltpu.sync_copy(ptile.at[pl.ds(lh_last * 8, 8)], pos_v.at[pl.ds(8, 8)])

        def coords(t):
            hb = hb0 + t
            lh = hb // 64
            rem = hb - lh * 64
            bh = rem // 2
            half = rem - bh * 2
            return lh, bh, half

        def idx_copy(t, idxb, sem):
            lh, bh, half = coords(t)
            return pltpu.make_async_copy(
                idx4.at[lh, bh, :, pl.ds(half * 64, 64)], idxb, sem)

        def gather_start(idxb, rows, sem):
            for ll in range(8):
                pltpu.make_async_copy(
                    table.at[idxb.at[ll]], rows.at[ll], sem).start()

        def gather_wait(idxb, rows, sem):
            for ll in range(8):
                pltpu.make_async_copy(
                    table.at[idxb.at[ll]], rows.at[ll], sem).wait()

        def out_start(t, trans, sem):
            lh, bh, half = coords(t)
            for ll in range(8):
                for dh in range(4):
                    pltpu.make_async_copy(
                        trans.at[pl.ds((ll * 4 + dh) * 8, 8), :],
                        out5.at[lh * 8 + ll, dh, bh, :,
                                pl.ds(half * 64, 64)],
                        sem).start()

        def out_wait(t, trans, sem):
            lh, bh, half = coords(t)
            for ll in range(8):
                for dh in range(4):
                    pltpu.make_async_copy(
                        trans.at[pl.ds((ll * 4 + dh) * 8, 8), :],
                        out5.at[lh * 8 + ll, dh, bh, :,
                                pl.ds(half * 64, 64)],
                        sem).wait()

        iota = lax.iota(jnp.int32, 16)
        dh0 = iota // 8
        dh1 = dh0 + 2
        dlv = lax.rem(iota, 8)

        def transpose_block(t, rows, trans):
            lh, _, _ = coords(t)
            lsel = (lh - lh_first) * 8
            for ll in range(8):
                prow = lsel + ll
                p0 = pos_v[prow, pl.ds(0, 16)]
                p1 = pos_v[prow, pl.ds(16, 16)]
                lvec = jnp.full((16,), ll, jnp.int32)

                row0 = lvec * 32 + iota
                row1 = row0 + 16

                def tok_body(tok, carry):
                    tvec = jnp.broadcast_to(tok.astype(jnp.int32), (16,))
                    v0 = rows[ll, tok, pl.ds(0, 16)] + p0
                    v1 = rows[ll, tok, pl.ds(16, 16)] + p1
                    plsc.store_scatter(trans, [row0, tvec], v0)
                    plsc.store_scatter(trans, [row1, tvec], v1)
                    return carry

                lax.fori_loop(0, 64, tok_body, 0)

        def loop_body(t, carry):
            idx_copy(t, idxb0, si0).start()
            idx_copy(t, idxb0, si0).wait()
            gather_start(idxb0, rows0, sg0)
            gather_wait(idxb0, rows0, sg0)
            transpose_block(t, rows0, trans0)
            out_start(t, trans0, so0)
            out_wait(t, trans0, so0)
            return carry

        lax.fori_loop(0, _BPW, loop_body, 0)

    return k


def kernel(inputs, table, pos):
    idx4 = (inputs.T.reshape(_LH, 8, _BH, 128)
            .transpose(0, 2, 1, 3).astype(jnp.int32))
    ptile = jnp.tile(pos, (2, 1))
    out5 = _make_kernel()(idx4, table, ptile)
    return out5.transpose(2, 4, 0, 1, 3).reshape(_B, _L, _D)


# speedup vs baseline: 1.0768x; 1.0768x over previous
import functools

import jax
import jax.numpy as jnp
from jax import lax
from jax.experimental import pallas as pl
from jax.experimental.pallas import tpu as pltpu
from jax.experimental.pallas import tpu_sc as plsc

_D = 32
_NC = 2
_NS = 16
_NW = _NC * _NS
_L = 200
_B = 4096
_LH = _L // 8
_BH = _B // 128
_NBLK = _LH * _BH * 2
_BPW = _NBLK // _NW


def _make_kernel():
    mesh = plsc.VectorSubcoreMesh(core_axis_name="c", subcore_axis_name="s")

    @functools.partial(
        pl.kernel,
        mesh=mesh,
        compiler_params=pltpu.CompilerParams(
            use_tc_tiling_on_sc=False, needs_layout_passes=False),
        out_type=jax.ShapeDtypeStruct((_L, 4, _BH, 8, 128), jnp.float32),
        scratch_types=[
            pltpu.VMEM((8, 64), jnp.int32),
            pltpu.VMEM((8, 64), jnp.int32),
            pltpu.VMEM((8, 64, _D), jnp.float32),
            pltpu.VMEM((8, 64, _D), jnp.float32),
            pltpu.VMEM((256, 64), jnp.float32),
            pltpu.VMEM((256, 64), jnp.float32),
            pltpu.VMEM((16, _D), jnp.float32),
            pltpu.SemaphoreType.DMA,
            pltpu.SemaphoreType.DMA,
            pltpu.SemaphoreType.DMA,
            pltpu.SemaphoreType.DMA,
            pltpu.SemaphoreType.DMA,
            pltpu.SemaphoreType.DMA,
        ],
    )
    def k(idx4, table, ptile, out5,
          idxb0, idxb1, rows0, rows1, trans0, trans1, pos_v,
          si0, si1, sg0, sg1, so0, so1):
        c = lax.axis_index("c")
        s = lax.axis_index("s")
        w = s * _NC + c
        hb0 = w * _BPW
        lh_first = hb0 // 64
        lh_last = (hb0 + _BPW - 1) // 64

        pltpu.sync_copy(ptile.at[pl.ds(lh_first * 8, 8)], pos_v.at[pl.ds(0, 8)])
        pltpu.sync_copy(ptile.at[pl.ds(lh_last * 8, 8)], pos_v.at[pl.ds(8, 8)])

        def coords(t):
            hb = hb0 + t
            lh = hb // 64
            rem = hb - lh * 64
            bh = rem // 2
            half = rem - bh * 2
            return lh, bh, half

        def idx_copy(t, idxb, sem):
            lh, bh, half = coords(t)
            return pltpu.make_async_copy(
                idx4.at[lh, bh, :, pl.ds(half * 64, 64)], idxb, sem)

        def gather_start(idxb, rows, sem):
            for ll in range(8):
                pltpu.make_async_copy(
                    table.at[idxb.at[ll]], rows.at[ll], sem).start()

        def gather_wait(idxb, rows, sem):
            for ll in range(8):
                pltpu.make_async_copy(
                    table.at[idxb.at[ll]], rows.at[ll], sem).wait()

        def out_start(t, trans, sem):
            lh, bh, half = coords(t)
            for ll in range(8):
                for dh in range(4):
                    pltpu.make_async_copy(
                        trans.at[pl.ds((ll * 4 + dh) * 8, 8), :],
                        out5.at[lh * 8 + ll, dh, bh, :,
                                pl.ds(half * 64, 64)],
                        sem).start()

        def out_wait(t, trans, sem):
            lh, bh, half = coords(t)
            for ll in range(8):
                for dh in range(4):
                    pltpu.make_async_copy(
                        trans.at[pl.ds((ll * 4 + dh) * 8, 8), :],
                        out5.at[lh * 8 + ll, dh, bh, :,
                                pl.ds(half * 64, 64)],
                        sem).wait()

        iota = lax.iota(jnp.int32, 16)
        dh0 = iota // 8
        dh1 = dh0 + 2
        dlv = lax.rem(iota, 8)

        def transpose_block(t, rows, trans):
            lh, _, _ = coords(t)
            lsel = (lh - lh_first) * 8
            for ll in range(8):
                prow = lsel + ll
                p0 = pos_v[prow, pl.ds(0, 16)]
                p1 = pos_v[prow, pl.ds(16, 16)]
                lvec = jnp.full((16,), ll, jnp.int32)

                row0 = lvec * 32 + iota
                row1 = row0 + 16

                def tok_body(tok, carry):
                    tvec = jnp.broadcast_to(tok.astype(jnp.int32), (16,))
                    v0 = rows[ll, tok, pl.ds(0, 16)] + p0
                    v1 = rows[ll, tok, pl.ds(16, 16)] + p1
                    plsc.store_scatter(trans, [row0, tvec], v0)
                    plsc.store_scatter(trans, [row1, tvec], v1)
                    return carry

                lax.fori_loop(0, 64, tok_body, 0)

        bufs = (
            (idxb0, rows0, trans0, si0, sg0, so0),
            (idxb1, rows1, trans1, si1, sg1, so1),
        )

        idx_copy(0, idxb0, si0).start()
        idx_copy(1, idxb1, si1).start()
        idx_copy(0, idxb0, si0).wait()
        gather_start(idxb0, rows0, sg0)

        def loop_body(it, carry):
            for sub in range(2):
                idxb, rows, trans, si, sg, so = bufs[sub]
                o_idxb, o_rows, _, o_si, o_sg, _ = bufs[1 - sub]
                t = it * 2 + sub

                @pl.when(t + 1 < _BPW)
                def _():
                    idx_copy(t + 1, o_idxb, o_si).wait()
                    gather_start(o_idxb, o_rows, o_sg)

                gather_wait(idxb, rows, sg)

                @pl.when(t >= 2)
                def _():
                    out_wait(t - 2, trans, so)

                transpose_block(t, rows, trans)
                out_start(t, trans, so)

                @pl.when(t + 2 < _BPW)
                def _():
                    idx_copy(t + 2, idxb, si).start()
            return carry

        lax.fori_loop(0, _BPW // 2, loop_body, 0)

        out_wait(_BPW - 2, trans0, so0)
        out_wait(_BPW - 1, trans1, so1)

    return k


def kernel(inputs, table, pos):
    idx4 = (inputs.T.reshape(_LH, 8, _BH, 128)
            .transpose(0, 2, 1, 3).astype(jnp.int32))
    ptile = jnp.tile(pos, (2, 1))
    out5 = _make_kernel()(idx4, table, ptile)
    return out5.transpose(2, 4, 0, 1, 3).reshape(_B, _L, _D)


# speedup vs baseline: 1.1787x; 1.0946x over previous
import functools

import jax
import jax.numpy as jnp
from jax import lax
from jax.experimental import pallas as pl
from jax.experimental.pallas import tpu as pltpu
from jax.experimental.pallas import tpu_sc as plsc

_D = 32
_NC = 2
_NS = 16
_NW = _NC * _NS
_L = 200
_B = 4096
_LH = _L // 8
_BH = _B // 128
_NBLK = _LH * _BH * 2
_BPW = _NBLK // _NW


def _make_kernel():
    mesh = plsc.VectorSubcoreMesh(core_axis_name="c", subcore_axis_name="s")

    @functools.partial(
        pl.kernel,
        mesh=mesh,
        compiler_params=pltpu.CompilerParams(
            use_tc_tiling_on_sc=False, needs_layout_passes=False),
        out_type=jax.ShapeDtypeStruct((_L, 4, _BH, 8, 128), jnp.float32),
        scratch_types=[
            pltpu.VMEM((4, 128), jnp.int32),
            pltpu.VMEM((4, 128), jnp.int32),
            pltpu.VMEM((4, 128, _D), jnp.float32),
            pltpu.VMEM((4, 128, _D), jnp.float32),
            pltpu.VMEM((128, 128), jnp.float32),
            pltpu.VMEM((128, 128), jnp.float32),
            pltpu.VMEM((16, _D), jnp.float32),
            pltpu.SemaphoreType.DMA,
            pltpu.SemaphoreType.DMA,
            pltpu.SemaphoreType.DMA,
            pltpu.SemaphoreType.DMA,
            pltpu.SemaphoreType.DMA,
            pltpu.SemaphoreType.DMA,
        ],
    )
    def k(idx4, table, ptile, out5,
          idxb0, idxb1, rows0, rows1, trans0, trans1, pos_v,
          si0, si1, sg0, sg1, so0, so1):
        c = lax.axis_index("c")
        s = lax.axis_index("s")
        w = s * _NC + c
        hb0 = w * _BPW
        lh_first = hb0 // 64
        lh_last = (hb0 + _BPW - 1) // 64

        pltpu.sync_copy(ptile.at[pl.ds(lh_first * 8, 8)], pos_v.at[pl.ds(0, 8)])
        pltpu.sync_copy(ptile.at[pl.ds(lh_last * 8, 8)], pos_v.at[pl.ds(8, 8)])

        def coords(t):
            hb = hb0 + t
            lh = hb // 64
            rem = hb - lh * 64
            bh = rem // 2
            ll0 = (rem - bh * 2) * 4
            return lh, bh, ll0

        def idx_copy(t, idxb, sem):
            lh, bh, ll0 = coords(t)
            return pltpu.make_async_copy(
                idx4.at[lh, bh, pl.ds(ll0, 4), :], idxb, sem)

        def gather_start(idxb, rows, sem):
            for j in range(4):
                pltpu.make_async_copy(
                    table.at[idxb.at[j]], rows.at[j], sem).start()

        def gather_wait(idxb, rows, sem):
            for j in range(4):
                pltpu.make_async_copy(
                    table.at[idxb.at[j]], rows.at[j], sem).wait()

        def out_start(t, trans, sem):
            lh, bh, ll0 = coords(t)
            for ll in range(4):
                for dh in range(4):
                    pltpu.make_async_copy(
                        trans.at[pl.ds((ll * 4 + dh) * 8, 8), :],
                        out5.at[lh * 8 + ll0 + ll, dh, bh, :, :],
                        sem).start()

        def out_wait(t, trans, sem):
            lh, bh, ll0 = coords(t)
            for ll in range(4):
                for dh in range(4):
                    pltpu.make_async_copy(
                        trans.at[pl.ds((ll * 4 + dh) * 8, 8), :],
                        out5.at[lh * 8 + ll0 + ll, dh, bh, :, :],
                        sem).wait()

        iota = lax.iota(jnp.int32, 16)

        def transpose_block(t, rows, trans):
            lh, _, ll0 = coords(t)
            lsel = (lh - lh_first) * 8 + ll0
            for ll in range(4):
                prow = lsel + ll
                p0 = pos_v[prow, pl.ds(0, 16)]
                p1 = pos_v[prow, pl.ds(16, 16)]
                row0 = iota + (ll * 32)
                row1 = row0 + 16

                @plsc.parallel_loop(0, 128, unroll=8)
                def _(tok):
                    tvec = jnp.broadcast_to(tok.astype(jnp.int32), (16,))
                    v0 = rows[ll, tok, pl.ds(0, 16)] + p0
                    v1 = rows[ll, tok, pl.ds(16, 16)] + p1
                    plsc.store_scatter(trans, [row0, tvec], v0)
                    plsc.store_scatter(trans, [row1, tvec], v1)

        bufs = (
            (idxb0, rows0, trans0, si0, sg0, so0),
            (idxb1, rows1, trans1, si1, sg1, so1),
        )

        idx_copy(0, idxb0, si0).start()
        idx_copy(1, idxb1, si1).start()
        idx_copy(0, idxb0, si0).wait()
        gather_start(idxb0, rows0, sg0)

        def loop_body(it, carry):
            for sub in range(2):
                idxb, rows, trans, si, sg, so = bufs[sub]
                o_idxb, o_rows, _, o_si, o_sg, _ = bufs[1 - sub]
                t = it * 2 + sub

                @pl.when(t + 1 < _BPW)
                def _():
                    idx_copy(t + 1, o_idxb, o_si).wait()
                    gather_start(o_idxb, o_rows, o_sg)

                gather_wait(idxb, rows, sg)

                @pl.when(t >= 2)
                def _():
                    out_wait(t - 2, trans, so)

                transpose_block(t, rows, trans)
                out_start(t, trans, so)

                @pl.when(t + 2 < _BPW)
                def _():
                    idx_copy(t + 2, idxb, si).start()
            return carry

        lax.fori_loop(0, _BPW // 2, loop_body, 0)

        out_wait(_BPW - 2, trans0, so0)
        out_wait(_BPW - 1, trans1, so1)

    return k


def kernel(inputs, table, pos):
    idx4 = (inputs.T.reshape(_LH, 8, _BH, 128)
            .transpose(0, 2, 1, 3).astype(jnp.int32))
    ptile = jnp.tile(pos, (2, 1))
    out5 = _make_kernel()(idx4, table, ptile)
    return out5.transpose(2, 4, 0, 1, 3).reshape(_B, _L, _D)


# speedup vs baseline: 1.2860x; 1.0910x over previous
import functools

import jax
import jax.numpy as jnp
from jax import lax
from jax.experimental import pallas as pl
from jax.experimental.pallas import tpu as pltpu
from jax.experimental.pallas import tpu_sc as plsc

_D = 32
_NC = 2
_NS = 16
_NW = _NC * _NS
_L = 200
_B = 4096
_LH = _L // 8
_BH = _B // 128
_NBLK = _LH * _BH * 2
_BPW = _NBLK // _NW


def _make_kernel():
    mesh = plsc.VectorSubcoreMesh(core_axis_name="c", subcore_axis_name="s")

    @functools.partial(
        pl.kernel,
        mesh=mesh,
        compiler_params=pltpu.CompilerParams(
            use_tc_tiling_on_sc=False, needs_layout_passes=False),
        out_type=jax.ShapeDtypeStruct((_L, 4, _BH, 8, 128), jnp.float32),
        scratch_types=[
            pltpu.VMEM((4, 128), jnp.int32),
            pltpu.VMEM((4, 128), jnp.int32),
            pltpu.VMEM((4, 128, _D), jnp.float32),
            pltpu.VMEM((4, 128, _D), jnp.float32),
            pltpu.VMEM((128, 128), jnp.float32),
            pltpu.VMEM((128, 128), jnp.float32),
            pltpu.VMEM((16, _D), jnp.float32),
            pltpu.SemaphoreType.DMA,
            pltpu.SemaphoreType.DMA,
            pltpu.SemaphoreType.DMA,
            pltpu.SemaphoreType.DMA,
            pltpu.SemaphoreType.DMA,
            pltpu.SemaphoreType.DMA,
        ],
    )
    def k(idx4, table, ptile, out5,
          idxb0, idxb1, rows0, rows1, trans0, trans1, pos_v,
          si0, si1, sg0, sg1, so0, so1):
        c = lax.axis_index("c")
        s = lax.axis_index("s")
        w = s * _NC + c
        hb0 = w * _BPW
        lh_first = hb0 // 64
        lh_last = (hb0 + _BPW - 1) // 64

        pltpu.sync_copy(ptile.at[pl.ds(lh_first * 8, 8)], pos_v.at[pl.ds(0, 8)])
        pltpu.sync_copy(ptile.at[pl.ds(lh_last * 8, 8)], pos_v.at[pl.ds(8, 8)])

        def coords(t):
            hb = hb0 + t
            lh = hb // 64
            rem = hb - lh * 64
            bh = rem // 2
            ll0 = (rem - bh * 2) * 4
            return lh, bh, ll0

        def idx_copy(t, idxb, sem):
            lh, bh, ll0 = coords(t)
            return pltpu.make_async_copy(
                idx4.at[lh, bh, pl.ds(ll0, 4), :], idxb, sem)

        def gather_start(idxb, rows, sem):
            for j in range(4):
                pltpu.make_async_copy(
                    table.at[idxb.at[j]], rows.at[j], sem).start()

        def gather_wait(idxb, rows, sem):
            for j in range(4):
                pltpu.make_async_copy(
                    table.at[idxb.at[j]], rows.at[j], sem).wait()

        def out_start(t, trans, sem):
            lh, bh, ll0 = coords(t)
            for ll in range(4):
                for dh in range(4):
                    pltpu.make_async_copy(
                        trans.at[pl.ds((ll * 4 + dh) * 8, 8), :],
                        out5.at[lh * 8 + ll0 + ll, dh, bh, :, :],
                        sem).start()

        def out_wait(t, trans, sem):
            lh, bh, ll0 = coords(t)
            for ll in range(4):
                for dh in range(4):
                    pltpu.make_async_copy(
                        trans.at[pl.ds((ll * 4 + dh) * 8, 8), :],
                        out5.at[lh * 8 + ll0 + ll, dh, bh, :, :],
                        sem).wait()

        iota = lax.iota(jnp.int32, 16)

        def transpose_block(t, rows, trans):
            lh, _, ll0 = coords(t)
            lsel = (lh - lh_first) * 8 + ll0
            ctoks = [iota + (g * 16) for g in range(8)]
            for ll in range(4):
                prow = lsel + ll
                pvec = jnp.broadcast_to(prow, (16,)).astype(jnp.int32)
                rll = rows.at[ll]
                rbase = ll * 32

                @plsc.parallel_loop(0, 32, unroll=4)
                def _(d):
                    dvec = jnp.broadcast_to(d.astype(jnp.int32), (16,))
                    psp = plsc.load_gather(pos_v, [pvec, dvec])
                    r = rbase + d
                    for g in range(8):
                        v = plsc.load_gather(rll, [ctoks[g], dvec])
                        trans[r, pl.ds(g * 16, 16)] = v + psp

        bufs = (
            (idxb0, rows0, trans0, si0, sg0, so0),
            (idxb1, rows1, trans1, si1, sg1, so1),
        )

        idx_copy(0, idxb0, si0).start()
        idx_copy(1, idxb1, si1).start()
        idx_copy(0, idxb0, si0).wait()
        gather_start(idxb0, rows0, sg0)

        def loop_body(it, carry):
            for sub in range(2):
                idxb, rows, trans, si, sg, so = bufs[sub]
                o_idxb, o_rows, _, o_si, o_sg, _ = bufs[1 - sub]
                t = it * 2 + sub

                @pl.when(t + 1 < _BPW)
                def _():
                    idx_copy(t + 1, o_idxb, o_si).wait()
                    gather_start(o_idxb, o_rows, o_sg)

                gather_wait(idxb, rows, sg)

                @pl.when(t >= 2)
                def _():
                    out_wait(t - 2, trans, so)

                transpose_block(t, rows, trans)
                out_start(t, trans, so)

                @pl.when(t + 2 < _BPW)
                def _():
                    idx_copy(t + 2, idxb, si).start()
            return carry

        lax.fori_loop(0, _BPW // 2, loop_body, 0)

        out_wait(_BPW - 2, trans0, so0)
        out_wait(_BPW - 1, trans1, so1)

    return k


def kernel(inputs, table, pos):
    idx4 = (inputs.T.reshape(_LH, 8, _BH, 128)
            .transpose(0, 2, 1, 3).astype(jnp.int32))
    ptile = jnp.tile(pos, (2, 1))
    out5 = _make_kernel()(idx4, table, ptile)
    return out5.transpose(2, 4, 0, 1, 3).reshape(_B, _L, _D)
